# baseline (speedup 1.0000x reference)
import functools

import jax
import jax.numpy as jnp
from jax import lax
from jax.experimental import pallas as pl
from jax.experimental.pallas import tpu as pltpu
from jax.experimental.pallas import tpu_sc as plsc

HIDDEN = 768
EPS = 1e-05
LANES = 16
NCH = HIDDEN // LANES
NC, NS = 2, 16
NW = NC * NS


def _rsqrt(a):
    i = lax.bitcast_convert_type(a, jnp.int32)
    y = lax.bitcast_convert_type(jnp.int32(0x5F3759DF) - (i >> 1), jnp.float32)
    half = a * 0.5
    for _ in range(3):
        y = y * (1.5 - half * y * y)
    return y


def _build(total, chunk):
    n_chunks = total // (NW * chunk)
    mesh = plsc.VectorSubcoreMesh(
        core_axis_name="c", subcore_axis_name="s",
        num_cores=NC, num_subcores=NS)

    @functools.partial(
        pl.kernel,
        mesh=mesh,
        out_type=jax.ShapeDtypeStruct((total, HIDDEN), jnp.float32),
        compiler_params=pltpu.CompilerParams(needs_layout_passes=False),
        scratch_types=[
            pltpu.VMEM((chunk,), jnp.int32),
            pltpu.VMEM((chunk, HIDDEN), jnp.float32),
            pltpu.VMEM((HIDDEN,), jnp.float32),
            pltpu.SemaphoreType.DMA,
        ],
    )
    def emb_ln(ids_hbm, table_hbm, w_hbm, out_hbm, idx_v, rows_v, w_v, sem):
        wid = lax.axis_index("s") * NC + lax.axis_index("c")
        base = wid * (n_chunks * chunk)
        pltpu.sync_copy(w_hbm, w_v)

        def chunk_body(ci, _):
            off = base + ci * chunk
            pltpu.sync_copy(ids_hbm.at[pl.ds(off, chunk)], idx_v)
            pltpu.async_copy(table_hbm.at[idx_v], rows_v, sem).wait()

            def row_body(r, _):
                def acc(c, carry):
                    s, q = carry
                    x = rows_v[r, pl.ds(c * LANES, LANES)]
                    return s + x, q + x * x
                zero = jnp.zeros((LANES,), jnp.float32)
                s, q = lax.fori_loop(0, NCH, acc, (zero, zero))
                mean = jnp.broadcast_to(jnp.sum(s), (LANES,)) * (1.0 / HIDDEN)
                msq = jnp.broadcast_to(jnp.sum(q), (LANES,)) * (1.0 / HIDDEN)
                scale = _rsqrt(msq - mean * mean + EPS)
                shift = mean * scale

                def norm(c, _):
                    x = rows_v[r, pl.ds(c * LANES, LANES)]
                    wv = w_v[pl.ds(c * LANES, LANES)]
                    rows_v[r, pl.ds(c * LANES, LANES)] = (x * scale - shift) * wv
                    return 0
                lax.fori_loop(0, NCH, norm, 0)
                return 0

            lax.fori_loop(0, chunk, row_body, 0)
            pltpu.sync_copy(rows_v, out_hbm.at[pl.ds(off, chunk)])
            return 0

        lax.fori_loop(0, n_chunks, chunk_body, 0)

    return emb_ln


@jax.jit
def kernel(input_ids, tok_embeddings, norm_weight):
    b, s = input_ids.shape
    total = b * s
    ids = input_ids.reshape(total).astype(jnp.int32)
    out = _build(total, 64)(ids, tok_embeddings, norm_weight)
    return out.reshape(b, s, HIDDEN)
